# baseline (speedup 1.0000x reference)
import jax
import jax.numpy as jnp
from jax.experimental import pallas as pl
from jax.experimental.pallas import tpu as pltpu

BN = 4096
NSUB = 8
LG = 128


def _fused_kernel(z_ref, ct_ref, cl_ref, loss_ref, c2_ref, iota_ref):
    i = pl.program_id(0)
    ct = ct_ref[...]

    @pl.when(i == 0)
    def _init():
        c2_ref[...] = jnp.sum(ct * ct, axis=0, keepdims=True) * 0.25
        iota_ref[...] = jax.lax.broadcasted_iota(
            jnp.int32, iota_ref.shape, 1).astype(jnp.float32)
        loss_ref[0, 0] = 0.0

    K = ct.shape[1]
    NG = K // LG
    SUB = BN // NSUB
    parts = []
    zs = [z_ref[j * SUB:(j + 1) * SUB, :] for j in range(NSUB)]
    ss = [jnp.dot(z, ct, preferred_element_type=jnp.float32) for z in zs]
    for j in range(NSUB):
        z = zs[j]
        s = ss[j]
        rm = c2_ref[:, :LG] + s[:, :LG]
        af = jnp.broadcast_to(iota_ref[:, :LG], rm.shape)
        for g in range(1, NG):
            cols = slice(g * LG, (g + 1) * LG)
            dg = c2_ref[:, cols] + s[:, cols]
            af = jnp.where(dg < rm, iota_ref[:, cols], af)
            rm = jnp.minimum(dg, rm)
        m = jnp.min(rm, axis=1, keepdims=True)
        amf = jnp.min(jnp.where(rm == m, af, float(K)), axis=1,
                      keepdims=True)
        cl_ref[j * SUB:(j + 1) * SUB, :] = amf.astype(jnp.int32)
        parts.append(jnp.sum(z * z))
        parts.append(jnp.sum(m))

    loss_ref[0, 0] += sum(parts)


def kernel(Z, centroids):
    N, D = Z.shape
    K, _ = centroids.shape
    grid = (N // BN,)
    ct = -2.0 * centroids.T

    cl, loss_sum = pl.pallas_call(
        _fused_kernel,
        grid=grid,
        in_specs=[
            pl.BlockSpec((BN, D), lambda i: (i, 0)),
            pl.BlockSpec((D, K), lambda i: (0, 0)),
        ],
        out_specs=[
            pl.BlockSpec((BN, 1), lambda i: (i, 0)),
            pl.BlockSpec(memory_space=pltpu.SMEM),
        ],
        out_shape=[
            jax.ShapeDtypeStruct((N, 1), jnp.int32),
            jax.ShapeDtypeStruct((1, 1), jnp.float32),
        ],
        scratch_shapes=[pltpu.VMEM((1, K), jnp.float32),
                        pltpu.VMEM((1, K), jnp.float32)],
    )(Z, ct)

    loss = loss_sum[0, 0] / N
    return (loss, cl.reshape(N))


# speedup vs baseline: 1.0350x; 1.0350x over previous
import jax
import jax.numpy as jnp
from jax.experimental import pallas as pl
from jax.experimental.pallas import tpu as pltpu

BN = 4096
NSUB = 16
LG = 128


def _fused_kernel(z_ref, ct_ref, cl_ref, loss_ref, c2_ref, iota_ref):
    i = pl.program_id(0)
    ct = ct_ref[...]

    @pl.when(i == 0)
    def _init():
        c2_ref[...] = jnp.sum(ct * ct, axis=0, keepdims=True) * 0.25
        iota_ref[...] = jax.lax.broadcasted_iota(
            jnp.int32, iota_ref.shape, 1).astype(jnp.float32)
        loss_ref[0, 0] = 0.0

    K = ct.shape[1]
    NG = K // LG
    SUB = BN // NSUB
    parts = []
    zs = [z_ref[j * SUB:(j + 1) * SUB, :] for j in range(NSUB)]
    ss = [jnp.dot(z, ct, preferred_element_type=jnp.float32) for z in zs]
    for j in range(NSUB):
        z = zs[j]
        s = ss[j]
        rm = c2_ref[:, :LG] + s[:, :LG]
        af = jnp.broadcast_to(iota_ref[:, :LG], rm.shape)
        for g in range(1, NG):
            cols = slice(g * LG, (g + 1) * LG)
            dg = c2_ref[:, cols] + s[:, cols]
            af = jnp.where(dg < rm, iota_ref[:, cols], af)
            rm = jnp.minimum(dg, rm)
        m = jnp.min(rm, axis=1, keepdims=True)
        amf = jnp.min(jnp.where(rm == m, af, float(K)), axis=1,
                      keepdims=True)
        cl_ref[j * SUB:(j + 1) * SUB, :] = amf.astype(jnp.int32)
        parts.append(jnp.sum(z * z))
        parts.append(jnp.sum(m))

    loss_ref[0, 0] += sum(parts)


def kernel(Z, centroids):
    N, D = Z.shape
    K, _ = centroids.shape
    grid = (N // BN,)
    ct = -2.0 * centroids.T

    cl, loss_sum = pl.pallas_call(
        _fused_kernel,
        grid=grid,
        in_specs=[
            pl.BlockSpec((BN, D), lambda i: (i, 0)),
            pl.BlockSpec((D, K), lambda i: (0, 0)),
        ],
        out_specs=[
            pl.BlockSpec((BN, 1), lambda i: (i, 0)),
            pl.BlockSpec(memory_space=pltpu.SMEM),
        ],
        out_shape=[
            jax.ShapeDtypeStruct((N, 1), jnp.int32),
            jax.ShapeDtypeStruct((1, 1), jnp.float32),
        ],
        scratch_shapes=[pltpu.VMEM((1, K), jnp.float32),
                        pltpu.VMEM((1, K), jnp.float32)],
    )(Z, ct)

    loss = loss_sum[0, 0] / N
    return (loss, cl.reshape(N))
